# baseline (speedup 1.0000x reference)
import functools

import jax
import jax.numpy as jnp
from jax import lax
from jax.experimental import pallas as pl
from jax.experimental.pallas import tpu as pltpu
from jax.experimental.pallas import tpu_sc as plsc

_VOCAB = 49152
_HIDDEN = 2048
_BATCH = 4
_SEQ = 4096
_NROWS = _BATCH * _SEQ
_NW = 32
_ROWS_PER_W = _NROWS // _NW
_S_PER_W = _SEQ // _NW
_NBUF = 4
_NGROUP = _S_PER_W // _NBUF


def _emb_lookup(idx, table):
    mesh = plsc.VectorSubcoreMesh(core_axis_name="c", subcore_axis_name="s")

    @functools.partial(
        pl.kernel,
        mesh=mesh,
        out_type=jax.ShapeDtypeStruct((_SEQ, _BATCH, _HIDDEN), jnp.float32),
        scratch_types=[
            pltpu.VMEM((_S_PER_W, _BATCH), jnp.int32),
        ]
        + [pltpu.VMEM((_BATCH, _HIDDEN), jnp.float32) for _ in range(_NBUF)]
        + [pltpu.SemaphoreType.DMA for _ in range(2 * _NBUF)],
    )
    def body(idx_hbm, table_hbm, out_hbm, idx_v, *scratch):
        bufs = scratch[:_NBUF]
        gsems = scratch[_NBUF:2 * _NBUF]
        ssems = scratch[2 * _NBUF:]
        wid = lax.axis_index("s") * 2 + lax.axis_index("c")
        s_base = wid * _S_PER_W

        s_start = pl.multiple_of(s_base, _S_PER_W)
        pltpu.sync_copy(idx_hbm.at[pl.ds(s_start, _S_PER_W), :], idx_v)

        def g_copy(chunk, b):
            return pltpu.make_async_copy(
                table_hbm.at[idx_v.at[chunk]],
                bufs[b], gsems[b])

        def s_copy(chunk, b):
            return pltpu.make_async_copy(
                bufs[b], out_hbm.at[s_base + chunk], ssems[b])

        def step(chunk, b, wait_prev_scatter, prefetch):
            g_copy(chunk, b).wait()
            s_copy(chunk, b).start()
            nb = (b + 2) % _NBUF
            if wait_prev_scatter:
                s_copy(chunk - 2, nb).wait()
            if prefetch:
                g_copy(chunk + 2, nb).start()

        g_copy(0, 0).start()
        g_copy(1, 1).start()

        step(0, 0, False, True)
        step(1, 1, False, True)
        step(2, 2, True, True)
        step(3, 3, True, True)

        def group_body(g, carry):
            c0 = g * _NBUF
            for b in range(_NBUF):
                step(c0 + b, b, True, True)
            return carry

        lax.fori_loop(1, _NGROUP - 1, group_body, 0)

        c0 = (_NGROUP - 1) * _NBUF
        step(c0 + 0, 0, True, True)
        step(c0 + 1, 1, True, True)
        step(c0 + 2, 2, True, False)
        step(c0 + 3, 3, True, False)
        s_copy(_S_PER_W - 2, 2).wait()
        s_copy(_S_PER_W - 1, 3).wait()

    return body(idx, table)


def kernel(input_ids, input_mask, token_embedding_weight):
    del input_mask
    idx_t = jnp.transpose(input_ids, (1, 0)).astype(jnp.int32)
    return _emb_lookup(idx_t, token_embedding_weight)


# speedup vs baseline: 1.0929x; 1.0929x over previous
import functools

import jax
import jax.numpy as jnp
from jax import lax
from jax.experimental import pallas as pl
from jax.experimental.pallas import tpu as pltpu
from jax.experimental.pallas import tpu_sc as plsc

_VOCAB = 49152
_HIDDEN = 2048
_BATCH = 4
_SEQ = 4096
_NW = 32
_SEQ_PER_W = _SEQ * _BATCH // _NW
_CHUNK = 16
_NBUF = 2
_NCHUNK = _SEQ_PER_W // _CHUNK
_NGROUP = _NCHUNK // _NBUF


def _emb_lookup(idx, table):
    mesh = plsc.VectorSubcoreMesh(core_axis_name="c", subcore_axis_name="s")

    @functools.partial(
        pl.kernel,
        mesh=mesh,
        out_type=jax.ShapeDtypeStruct((_SEQ, _BATCH, _HIDDEN), jnp.float32),
        scratch_types=[
            pltpu.VMEM((_SEQ_PER_W,), jnp.int32),
            pltpu.VMEM((_CHUNK, _HIDDEN), jnp.float32),
            pltpu.VMEM((_CHUNK, _HIDDEN), jnp.float32),
            pltpu.SemaphoreType.DMA,
            pltpu.SemaphoreType.DMA,
        ],
    )
    def body(idx_hbm, table_hbm, out_hbm, idx_v, buf0, buf1, sem0, sem1):
        wid = lax.axis_index("s") * 2 + lax.axis_index("c")
        b = wid & (_BATCH - 1)
        s_base = (wid >> 2) * _SEQ_PER_W
        s_start = pl.multiple_of(s_base, _SEQ_PER_W)
        pltpu.sync_copy(idx_hbm.at[b, pl.ds(s_start, _SEQ_PER_W)], idx_v)
        bufs = (buf0, buf1)
        sems = (sem0, sem1)

        def gather(chunk, slot):
            return pltpu.make_async_copy(
                table_hbm.at[idx_v.at[pl.ds(chunk * _CHUNK, _CHUNK)]],
                bufs[slot], sems[slot])

        def drain(chunk, slot):
            gather(chunk, slot).wait()
            pltpu.sync_copy(
                bufs[slot],
                out_hbm.at[pl.ds(s_start + chunk * _CHUNK, _CHUNK), b])

        for slot in range(_NBUF):
            gather(slot, slot).start()

        def group_body(g, carry):
            for slot in range(_NBUF):
                chunk = g * _NBUF + slot
                drain(chunk, slot)
                gather(chunk + _NBUF, slot).start()
            return carry

        lax.fori_loop(0, _NGROUP - 1, group_body, 0)

        for slot in range(_NBUF):
            drain((_NGROUP - 1) * _NBUF + slot, slot)

    return body(idx, table)


def kernel(input_ids, input_mask, token_embedding_weight):
    del input_mask
    return _emb_lookup(input_ids.astype(jnp.int32), token_embedding_weight)
